# baseline (speedup 1.0000x reference)
import functools

import jax
import jax.numpy as jnp
from jax import lax
from jax.experimental import pallas as pl
from jax.experimental.pallas import tpu as pltpu
from jax.experimental.pallas import tpu_sc as plsc

P_TOTAL = 500000
CHUNK = 2000
NCHUNK = P_TOTAL // CHUNK
GROUPS = CHUNK // 16
NW = 32
TSTEPS = (NCHUNK + NW - 1) // NW

_F32_MAX = 3.4028234663852886e38


def _nan_to_num(q):
    q = jnp.where(q != q, 0.0, q)
    return jnp.minimum(jnp.maximum(q, -_F32_MAX), _F32_MAX)


def _cog_body(pos_hbm, s_hbm, opos_hbm, osc_hbm, pos_v, s_v, opos_v, osc_v):
    wid = lax.axis_index("s") * 2 + lax.axis_index("c")
    iota = lax.iota(jnp.int32, 16)
    idx4 = iota * 4
    idx12 = iota * 12
    idx3 = iota * 3

    def group_body(g, carry):
        sb = g * 64
        pb = g * 192
        ob = g * 48
        s = [plsc.load_gather(s_v, [sb + idx4 + j]) for j in range(4)]
        px = [plsc.load_gather(pos_v, [pb + idx12 + 3 * j]) for j in range(4)]
        py = [plsc.load_gather(pos_v, [pb + idx12 + 3 * j + 1]) for j in range(4)]
        pz = [plsc.load_gather(pos_v, [pb + idx12 + 3 * j + 2]) for j in range(4)]

        ssum = (s[0] + s[1]) + (s[2] + s[3])
        numx = (s[0] * px[0] + s[1] * px[1]) + (s[2] * px[2] + s[3] * px[3])
        numy = (s[0] * py[0] + s[1] * py[1]) + (s[2] * py[2] + s[3] * py[3])
        numz = (s[0] * pz[0] + s[1] * pz[1]) + (s[2] * pz[2] + s[3] * pz[3])
        cnt = ((jnp.where(s[0] > 0, 1.0, 0.0) + jnp.where(s[1] > 0, 1.0, 0.0))
               + (jnp.where(s[2] > 0, 1.0, 0.0) + jnp.where(s[3] > 0, 1.0, 0.0)))

        qx = _nan_to_num(numx / ssum)
        qy = _nan_to_num(numy / ssum)
        qz = _nan_to_num(numz / ssum)
        qs = _nan_to_num(ssum / cnt)

        plsc.store_scatter(opos_v, [ob + idx3], qx)
        plsc.store_scatter(opos_v, [ob + idx3 + 1], qy)
        plsc.store_scatter(opos_v, [ob + idx3 + 2], qz)
        osc_v[pl.ds(g * 16, 16)] = qs
        return carry

    for t in range(TSTEPS):
        ci = wid + NW * t

        @pl.when(ci < NCHUNK)
        def _process():
            base = ci * CHUNK
            pltpu.sync_copy(s_hbm.at[pl.ds(base * 4, CHUNK * 4)], s_v)
            pltpu.sync_copy(pos_hbm.at[pl.ds(base * 12, CHUNK * 12)], pos_v)
            lax.fori_loop(0, GROUPS, group_body, 0)
            pltpu.sync_copy(opos_v, opos_hbm.at[pl.ds(base * 3, CHUNK * 3)])
            pltpu.sync_copy(osc_v, osc_hbm.at[pl.ds(base, CHUNK)])


@functools.partial(
    pl.kernel,
    out_type=(
        jax.ShapeDtypeStruct((P_TOTAL * 3,), jnp.float32),
        jax.ShapeDtypeStruct((P_TOTAL,), jnp.float32),
    ),
    mesh=plsc.VectorSubcoreMesh(core_axis_name="c", subcore_axis_name="s"),
    scratch_types=(
        pltpu.VMEM((CHUNK * 12,), jnp.float32),
        pltpu.VMEM((CHUNK * 4,), jnp.float32),
        pltpu.VMEM((CHUNK * 3,), jnp.float32),
        pltpu.VMEM((CHUNK,), jnp.float32),
    ),
    compiler_params=pltpu.CompilerParams(needs_layout_passes=False),
)
def _cog_kernel(pos_hbm, s_hbm, opos_hbm, osc_hbm, pos_v, s_v, opos_v, osc_v):
    _cog_body(pos_hbm, s_hbm, opos_hbm, osc_hbm, pos_v, s_v, opos_v, osc_v)


def kernel(position, scores):
    pos_flat = position.reshape(-1)
    opos, osc = _cog_kernel(pos_flat, scores)
    return opos.reshape(P_TOTAL, 3), osc
